# Initial kernel scaffold; baseline (speedup 1.0000x reference)
import jax
import jax.numpy as jnp
from jax.experimental import pallas as pl


def kernel(x, W):
    raise NotImplementedError("write your pallas kernel here")


# speedup vs baseline: 1.5615x; 1.5615x over previous
import jax
import jax.numpy as jnp
from jax.experimental import pallas as pl
from jax.experimental.pallas import tpu as pltpu

_NEXP = 64
_TOPK = 2
_BT = 512


def _gate_block(x_ref, w_ref, wout_ref, iout_ref):
    x = x_ref[...]
    w = w_ref[...]
    logits = jax.lax.dot_general(
        x, w, (((1,), (1,)), ((), ())),
        preferred_element_type=jnp.float32)

    ids = jax.lax.broadcasted_iota(jnp.int32, logits.shape, 1)
    m1 = jnp.max(logits, axis=1, keepdims=True)
    denom = jnp.sum(jnp.exp(logits - m1), axis=1, keepdims=True)
    big = jnp.int32(_NEXP)
    i1 = jnp.min(jnp.where(logits == m1, ids, big), axis=1, keepdims=True)
    masked = jnp.where(ids == i1, -jnp.inf, logits)
    m2 = jnp.max(masked, axis=1, keepdims=True)
    i2 = jnp.min(jnp.where(masked == m2, ids, big), axis=1, keepdims=True)

    w1 = jnp.exp(m1 - m1) / denom
    w2 = jnp.exp(m2 - m1) / denom

    slot = jax.lax.broadcasted_iota(jnp.int32, (x.shape[0], _TOPK), 1)
    wout_ref[...] = jnp.where(slot == 0, w1, w2)
    iout_ref[...] = jnp.where(slot == 0, i1, i2)


def kernel(x, W):
    ntok, dim = x.shape
    grid = (ntok // _BT,)
    weights, indices = pl.pallas_call(
        _gate_block,
        grid=grid,
        in_specs=[
            pl.BlockSpec((_BT, dim), lambda i: (i, 0)),
            pl.BlockSpec((_NEXP, dim), lambda i: (0, 0)),
        ],
        out_specs=[
            pl.BlockSpec((_BT, _TOPK), lambda i: (i, 0)),
            pl.BlockSpec((_BT, _TOPK), lambda i: (i, 0)),
        ],
        out_shape=[
            jax.ShapeDtypeStruct((ntok, _TOPK), jnp.float32),
            jax.ShapeDtypeStruct((ntok, _TOPK), jnp.int32),
        ],
        compiler_params=pltpu.CompilerParams(
            dimension_semantics=("arbitrary",),
        ),
    )(x, W)
    return (weights, indices)


# speedup vs baseline: 1.7817x; 1.1410x over previous
import jax
import jax.numpy as jnp
from jax.experimental import pallas as pl
from jax.experimental.pallas import tpu as pltpu

_NEXP = 64
_TOPK = 2
_BT = 1024


def _gate_block(x_ref, w_ref, wout_ref, iout_ref):
    x = x_ref[...]
    w = w_ref[...]
    logits = jax.lax.dot_general(
        x, w, (((1,), (1,)), ((), ())),
        preferred_element_type=jnp.float32)

    ids = jax.lax.broadcasted_iota(jnp.int32, logits.shape, 1)
    m1 = jnp.max(logits, axis=1, keepdims=True)
    denom = jnp.sum(jnp.exp(logits - m1), axis=1, keepdims=True)
    big = jnp.int32(_NEXP)
    i1 = jnp.min(jnp.where(logits == m1, ids, big), axis=1, keepdims=True)
    masked = jnp.where(ids == i1, -jnp.inf, logits)
    m2 = jnp.max(masked, axis=1, keepdims=True)
    i2 = jnp.min(jnp.where(masked == m2, ids, big), axis=1, keepdims=True)

    w1 = jnp.exp(m1 - m1) / denom
    w2 = jnp.exp(m2 - m1) / denom

    slot = jax.lax.broadcasted_iota(jnp.int32, (x.shape[0], _TOPK), 1)
    wout_ref[...] = jnp.where(slot == 0, w1, w2)
    iout_ref[...] = jnp.where(slot == 0, i1, i2)


def kernel(x, W):
    ntok, dim = x.shape
    grid = (ntok // _BT,)
    weights, indices = pl.pallas_call(
        _gate_block,
        grid=grid,
        in_specs=[
            pl.BlockSpec((_BT, dim), lambda i: (i, 0)),
            pl.BlockSpec((_NEXP, dim), lambda i: (0, 0)),
        ],
        out_specs=[
            pl.BlockSpec((_BT, _TOPK), lambda i: (i, 0)),
            pl.BlockSpec((_BT, _TOPK), lambda i: (i, 0)),
        ],
        out_shape=[
            jax.ShapeDtypeStruct((ntok, _TOPK), jnp.float32),
            jax.ShapeDtypeStruct((ntok, _TOPK), jnp.int32),
        ],
        compiler_params=pltpu.CompilerParams(
            dimension_semantics=("arbitrary",),
        ),
    )(x, W)
    return (weights, indices)


# speedup vs baseline: 1.7940x; 1.0069x over previous
import jax
import jax.numpy as jnp
from jax.experimental import pallas as pl
from jax.experimental.pallas import tpu as pltpu

_NEXP = 64
_TOPK = 2
_BT = 2048


def _gate_block(x_ref, w_ref, wout_ref, iout_ref):
    x = x_ref[...]
    w = w_ref[...]
    logits = jax.lax.dot_general(
        x, w, (((1,), (1,)), ((), ())),
        preferred_element_type=jnp.float32)

    ids = jax.lax.broadcasted_iota(jnp.int32, logits.shape, 1)
    m1 = jnp.max(logits, axis=1, keepdims=True)
    denom = jnp.sum(jnp.exp(logits - m1), axis=1, keepdims=True)
    big = jnp.int32(_NEXP)
    i1 = jnp.min(jnp.where(logits == m1, ids, big), axis=1, keepdims=True)
    masked = jnp.where(ids == i1, -jnp.inf, logits)
    m2 = jnp.max(masked, axis=1, keepdims=True)
    i2 = jnp.min(jnp.where(masked == m2, ids, big), axis=1, keepdims=True)

    w1 = jnp.exp(m1 - m1) / denom
    w2 = jnp.exp(m2 - m1) / denom

    slot = jax.lax.broadcasted_iota(jnp.int32, (x.shape[0], _TOPK), 1)
    wout_ref[...] = jnp.where(slot == 0, w1, w2)
    iout_ref[...] = jnp.where(slot == 0, i1, i2)


def kernel(x, W):
    ntok, dim = x.shape
    grid = (ntok // _BT,)
    weights, indices = pl.pallas_call(
        _gate_block,
        grid=grid,
        in_specs=[
            pl.BlockSpec((_BT, dim), lambda i: (i, 0)),
            pl.BlockSpec((_NEXP, dim), lambda i: (0, 0)),
        ],
        out_specs=[
            pl.BlockSpec((_BT, _TOPK), lambda i: (i, 0)),
            pl.BlockSpec((_BT, _TOPK), lambda i: (i, 0)),
        ],
        out_shape=[
            jax.ShapeDtypeStruct((ntok, _TOPK), jnp.float32),
            jax.ShapeDtypeStruct((ntok, _TOPK), jnp.int32),
        ],
        compiler_params=pltpu.CompilerParams(
            dimension_semantics=("arbitrary",),
        ),
    )(x, W)
    return (weights, indices)
